# baseline (speedup 1.0000x reference)
import jax
import jax.numpy as jnp
import numpy as np
from jax import lax
from jax.experimental import pallas as pl
from jax.experimental.pallas import tpu as pltpu
from jax.experimental.pallas import tpu_sc as plsc

NC = 2
NS = 16
NW = NC * NS
LANES = 16

B = 4096
K = 50
KP = 56
CONV = 256
CONVW = CONV // 2
NCH = CONV // LANES
ROWS_PW = B // NW
NBUF = 4
VOCAB = 100000


def _bag_body(idx_hbm, vals_hbm, table_hbm, y_hbm,
              idx_v, vals_v, bufs, ystage, sems):
    c = lax.axis_index("c")
    s = lax.axis_index("s")
    wid = s * NC + c
    rbase = wid * ROWS_PW

    pltpu.sync_copy(idx_hbm.at[pl.ds(rbase, ROWS_PW), :], idx_v)
    pltpu.sync_copy(vals_hbm.at[pl.ds(rbase * KP, ROWS_PW * KP)], vals_v)

    def issue(r, b):
        pltpu.async_copy(table_hbm.at[idx_v.at[r]], bufs[b], sems[b])

    def wait(r, b):
        pltpu.make_async_copy(table_hbm.at[idx_v.at[r]],
                              bufs[b], sems[b]).wait()

    for b in range(NBUF - 1):
        issue(b, b)

    def accum_row(r, buf):
        def jbody(j, acc):
            w = plsc.load_gather(
                vals_v, [jnp.full((LANES,), r * KP + j, jnp.int32)])
            out = []
            for ci in range(NCH // 2):
                chunk = plsc.bitcast(buf[j, pl.ds(ci * LANES, LANES)],
                                     jnp.bfloat16)
                a, bb = plsc.unpack(chunk, format=plsc.PackFormat.INTERLEAVED,
                                    preferred_element_type=jnp.float32)
                out.append(acc[2 * ci] + w * a)
                out.append(acc[2 * ci + 1] + w * bb)
            return tuple(out)

        acc = lax.fori_loop(
            0, KP, jbody,
            tuple(jnp.zeros((LANES,), jnp.float32) for _ in range(NCH)),
            unroll=2)
        for ci in range(NCH // 2):
            ystage[r, pl.ds(ci * 2 * LANES, LANES)] = acc[2 * ci]
            ystage[r, pl.ds(ci * 2 * LANES + LANES, LANES)] = acc[2 * ci + 1]

    def gbody(gg, carry):
        for b in range(NBUF):
            r = NBUF * gg + b

            @pl.when(r + NBUF - 1 < ROWS_PW)
            def _issue_next(r=r, b=b):
                issue(r + NBUF - 1, (b + NBUF - 1) % NBUF)

            wait(r, b)
            accum_row(r, bufs[b])
        return carry

    lax.fori_loop(0, ROWS_PW // NBUF, gbody, None)

    pltpu.sync_copy(ystage, y_hbm.at[pl.ds(rbase, ROWS_PW), :])


@jax.jit
def _bag(idx_p, vals_p, table_pk):
    mesh = plsc.VectorSubcoreMesh(core_axis_name="c", subcore_axis_name="s")

    def body(idx_hbm, vals_hbm, table_hbm, y_hbm, *scratch):
        _bag_body(idx_hbm, vals_hbm, table_hbm, y_hbm,
                  scratch[0], scratch[1], scratch[2:2 + NBUF],
                  scratch[2 + NBUF], scratch[3 + NBUF:])

    return pl.kernel(
        body,
        out_type=jax.ShapeDtypeStruct((B, CONV), jnp.float32),
        mesh=mesh,
        scratch_types=(
            [pltpu.VMEM((ROWS_PW, KP), jnp.int32),
             pltpu.VMEM((ROWS_PW * KP,), jnp.float32)]
            + [pltpu.VMEM((KP, CONVW), jnp.int32) for _ in range(NBUF)]
            + [pltpu.VMEM((ROWS_PW, CONV), jnp.float32)]
            + [pltpu.SemaphoreType.DMA for _ in range(NBUF)]
        ),
        compiler_params=pltpu.CompilerParams(needs_layout_passes=False),
    )(idx_p, vals_p, table_pk)


def _pack_table(table):
    u = lax.bitcast_convert_type(table, jnp.uint32)

    def rnd(x):
        return x + 0x7FFF + ((x >> 16) & 1)

    a = rnd(u[:, 0::2])
    b = rnd(u[:, 1::2])
    word = (a >> 16) | (b & jnp.uint32(0xFFFF0000))
    return lax.bitcast_convert_type(word, jnp.int32)


def _tail_body(y_ref, wmt_ref, bm_ref, wit_ref, bi_ref, rad_ref, out_ref):
    y = y_ref[...]
    inv1 = lax.rsqrt(jnp.maximum(jnp.sum(y * y, axis=1, keepdims=True),
                                 1e-24))
    h = y * inv1
    h = jnp.dot(h, wmt_ref[...], preferred_element_type=jnp.float32,
                precision=lax.Precision.HIGHEST) + bm_ref[...]
    h = jnp.where(h >= 0, h, 0.01 * h)
    h = jnp.dot(h, wit_ref[...], preferred_element_type=jnp.float32,
                precision=lax.Precision.HIGHEST) + bi_ref[...]
    inv2 = lax.rsqrt(jnp.maximum(jnp.sum(h * h, axis=1, keepdims=True),
                                 1e-24))
    out_ref[...] = (rad_ref[0, 0] * inv2) * h


@jax.jit
def _tail(y, wmt, bm, wit, bi, rad):
    BR = 1024
    return pl.pallas_call(
        _tail_body,
        grid=(B // BR,),
        in_specs=[
            pl.BlockSpec((BR, CONV), lambda i: (i, 0)),
            pl.BlockSpec(wmt.shape, lambda i: (0, 0)),
            pl.BlockSpec(bm.shape, lambda i: (0, 0)),
            pl.BlockSpec(wit.shape, lambda i: (0, 0)),
            pl.BlockSpec(bi.shape, lambda i: (0, 0)),
            pl.BlockSpec(rad.shape, lambda i: (0, 0)),
        ],
        out_specs=pl.BlockSpec((BR, wit.shape[1]), lambda i: (i, 0)),
        out_shape=jax.ShapeDtypeStruct((B, wit.shape[1]), jnp.float32),
    )(y, wmt, bm, wit, bi, rad)


_PERM = np.asarray(
    [32 * c + 2 * t + o
     for c in range(NCH // 2) for o in (0, 1) for t in range(LANES)],
    dtype=np.int32)


def kernel(indices, offsets, vals, table, W_mid, b_mid, W_i, b_i, radius_w):
    del offsets
    idx2 = indices.reshape(B, K).astype(jnp.int32)
    v2 = vals.reshape(B, K)
    idx_p = jnp.pad(idx2, ((0, 0), (0, KP - K)))
    vals_p = jnp.pad(v2, ((0, 0), (0, KP - K))).reshape(-1)
    y = _bag(idx_p, vals_p, _pack_table(table))
    return _tail(y, W_mid.T[_PERM, :], b_mid.reshape(1, -1), W_i.T,
                 b_i.reshape(1, -1), radius_w)


# speedup vs baseline: 3.9693x; 3.9693x over previous
import jax
import jax.numpy as jnp
import numpy as np
from jax import lax
from jax.experimental import pallas as pl
from jax.experimental.pallas import tpu as pltpu
from jax.experimental.pallas import tpu_sc as plsc

NC = 2
NS = 16
NW = NC * NS
LANES = 16

B = 4096
K = 50
KP = 56
CONV = 256
CONVW = CONV // 2
NCH = CONV // LANES
ROWS_PW = B // NW
NBUF = 4
VOCAB = 100000


def _bag_body(idx_hbm, vals_hbm, table_hbm, y_hbm,
              idx_v, vals_v, bufs, ystage, sems):
    c = lax.axis_index("c")
    s = lax.axis_index("s")
    wid = s * NC + c
    rbase = wid * ROWS_PW

    pltpu.sync_copy(idx_hbm.at[pl.ds(rbase, ROWS_PW), :], idx_v)
    pltpu.sync_copy(vals_hbm.at[pl.ds(rbase * KP, ROWS_PW * KP)], vals_v)

    def issue(r, b):
        pltpu.async_copy(table_hbm.at[idx_v.at[r]], bufs[b], sems[b])

    def wait(r, b):
        pltpu.make_async_copy(table_hbm.at[idx_v.at[r]],
                              bufs[b], sems[b]).wait()

    for b in range(NBUF - 1):
        issue(b, b)

    def accum_row(r, buf):
        def jbody(j, acc):
            w = plsc.load_gather(
                vals_v, [jnp.full((LANES,), r * KP + j, jnp.int32)])
            out = []
            for ci in range(NCH // 2):
                chunk = plsc.bitcast(buf[j, pl.ds(ci * LANES, LANES)],
                                     jnp.bfloat16)
                a, bb = plsc.unpack(chunk, format=plsc.PackFormat.INTERLEAVED,
                                    preferred_element_type=jnp.float32)
                out.append(acc[2 * ci] + w * a)
                out.append(acc[2 * ci + 1] + w * bb)
            return tuple(out)

        acc = lax.fori_loop(
            0, KP, jbody,
            tuple(jnp.zeros((LANES,), jnp.float32) for _ in range(NCH)),
            unroll=2)
        for ci in range(NCH // 2):
            ystage[r, pl.ds(ci * LANES, LANES)] = acc[2 * ci]
            ystage[r, pl.ds(CONVW + ci * LANES, LANES)] = acc[2 * ci + 1]

    def gbody(gg, carry):
        for b in range(NBUF):
            r = NBUF * gg + b

            @pl.when(r + NBUF - 1 < ROWS_PW)
            def _issue_next(r=r, b=b):
                issue(r + NBUF - 1, (b + NBUF - 1) % NBUF)

            wait(r, b)
            accum_row(r, bufs[b])
        return carry

    lax.fori_loop(0, ROWS_PW // NBUF, gbody, None)

    pltpu.sync_copy(ystage, y_hbm.at[pl.ds(rbase, ROWS_PW), :])


@jax.jit
def _bag(idx_p, vals_p, table_pk):
    mesh = plsc.VectorSubcoreMesh(core_axis_name="c", subcore_axis_name="s")

    def body(idx_hbm, vals_hbm, table_hbm, y_hbm, *scratch):
        _bag_body(idx_hbm, vals_hbm, table_hbm, y_hbm,
                  scratch[0], scratch[1], scratch[2:2 + NBUF],
                  scratch[2 + NBUF], scratch[3 + NBUF:])

    return pl.kernel(
        body,
        out_type=jax.ShapeDtypeStruct((B, CONV), jnp.float32),
        mesh=mesh,
        scratch_types=(
            [pltpu.VMEM((ROWS_PW, KP), jnp.int32),
             pltpu.VMEM((ROWS_PW * KP,), jnp.float32)]
            + [pltpu.VMEM((KP, CONVW), jnp.int32) for _ in range(NBUF)]
            + [pltpu.VMEM((ROWS_PW, CONV), jnp.float32)]
            + [pltpu.SemaphoreType.DMA for _ in range(NBUF)]
        ),
        compiler_params=pltpu.CompilerParams(needs_layout_passes=False),
    )(idx_p, vals_p, table_pk)


def _pack_table(table):
    u = lax.bitcast_convert_type(table, jnp.uint32)

    def rnd(x):
        return x + 0x7FFF + ((x >> 16) & 1)

    a = rnd(u[:, :CONVW])
    b = rnd(u[:, CONVW:])
    word = (a >> 16) | (b & jnp.uint32(0xFFFF0000))
    return lax.bitcast_convert_type(word, jnp.int32)


def _tail_body(y_ref, wmt_ref, bm_ref, wit_ref, bi_ref, rad_ref, out_ref):
    y = y_ref[...]
    inv1 = lax.rsqrt(jnp.maximum(jnp.sum(y * y, axis=1, keepdims=True),
                                 1e-24))
    h = y * inv1
    h = jnp.dot(h, wmt_ref[...], preferred_element_type=jnp.float32,
                precision=lax.Precision.HIGHEST) + bm_ref[...]
    h = jnp.where(h >= 0, h, 0.01 * h)
    h = jnp.dot(h, wit_ref[...], preferred_element_type=jnp.float32,
                precision=lax.Precision.HIGHEST) + bi_ref[...]
    inv2 = lax.rsqrt(jnp.maximum(jnp.sum(h * h, axis=1, keepdims=True),
                                 1e-24))
    out_ref[...] = (rad_ref[0, 0] * inv2) * h


@jax.jit
def _tail(y, wmt, bm, wit, bi, rad):
    BR = 1024
    return pl.pallas_call(
        _tail_body,
        grid=(B // BR,),
        in_specs=[
            pl.BlockSpec((BR, CONV), lambda i: (i, 0)),
            pl.BlockSpec(wmt.shape, lambda i: (0, 0)),
            pl.BlockSpec(bm.shape, lambda i: (0, 0)),
            pl.BlockSpec(wit.shape, lambda i: (0, 0)),
            pl.BlockSpec(bi.shape, lambda i: (0, 0)),
            pl.BlockSpec(rad.shape, lambda i: (0, 0)),
        ],
        out_specs=pl.BlockSpec((BR, wit.shape[1]), lambda i: (i, 0)),
        out_shape=jax.ShapeDtypeStruct((B, wit.shape[1]), jnp.float32),
    )(y, wmt, bm, wit, bi, rad)


def kernel(indices, offsets, vals, table, W_mid, b_mid, W_i, b_i, radius_w):
    del offsets
    idx2 = indices.reshape(B, K).astype(jnp.int32)
    v2 = vals.reshape(B, K)
    idx_p = jnp.pad(idx2, ((0, 0), (0, KP - K)))
    vals_p = jnp.pad(v2, ((0, 0), (0, KP - K))).reshape(-1)
    y = _bag(idx_p, vals_p, _pack_table(table))
    return _tail(y, W_mid.T, b_mid.reshape(1, -1), W_i.T,
                 b_i.reshape(1, -1), radius_w)
